# Initial kernel scaffold; baseline (speedup 1.0000x reference)
import jax
import jax.numpy as jnp
from jax.experimental import pallas as pl


def kernel(x, edge_index, batch, W1, b1, W2, b2):
    raise NotImplementedError("write your pallas kernel here")


# speedup vs baseline: 27.6409x; 27.6409x over previous
import functools

import jax
import jax.numpy as jnp
from jax import lax
from jax.experimental import pallas as pl
from jax.experimental.pallas import tpu as pltpu
from jax.experimental.pallas import tpu_sc as plsc

NC = 2
NS = 16
NW = NC * NS


def _sc_scalar_scatter(E, N, NP):
    EW = E // NW
    CH = 80
    NCH = EW // CH
    TS = NP // NS
    mesh = plsc.VectorSubcoreMesh(core_axis_name="c", subcore_axis_name="s")

    @functools.partial(
        pl.kernel,
        out_type=jax.ShapeDtypeStruct((NC, 1, NP), jnp.float32),
        mesh=mesh,
        scratch_types=[
            pltpu.VMEM((NCH, CH), jnp.int32),
            pltpu.VMEM((NCH, CH), jnp.int32),
            pltpu.VMEM((N,), jnp.float32),
            pltpu.VMEM((NCH, CH), jnp.float32),
            pltpu.VMEM_SHARED((NP,), jnp.float32),
            pltpu.SemaphoreType.DMA,
        ],
        compiler_params=pltpu.CompilerParams(
            use_tc_tiling_on_sc=False, needs_layout_passes=False),
    )
    def k(src_hbm, dst_hbm, val_hbm, zvec_hbm, out_hbm,
          src_buf, dst_buf, val_tab, vals_buf, acc, sem):
        c = lax.axis_index("c")
        s = lax.axis_index("s")
        w = c * NS + s
        pltpu.sync_copy(zvec_hbm.at[pl.ds(s * TS, TS)], acc.at[pl.ds(s * TS, TS)])
        pltpu.sync_copy(val_hbm, val_tab)
        pltpu.sync_copy(src_hbm.at[w], src_buf)
        pltpu.sync_copy(dst_hbm.at[w], dst_buf)

        def gather_body(j, carry):
            for u in range(CH // 16):
                idx = src_buf[j, pl.ds(u * 16, 16)]
                vals_buf[j, pl.ds(u * 16, 16)] = plsc.load_gather(val_tab, [idx])
            return carry

        lax.fori_loop(0, NCH, gather_body, 0)
        plsc.subcore_barrier()

        def scatter_body(j, carry):
            pltpu.sync_copy(vals_buf.at[j], acc.at[dst_buf.at[j]], add=True)
            return carry

        lax.fori_loop(0, NCH, scatter_body, 0)
        plsc.subcore_barrier()
        pltpu.sync_copy(acc.at[pl.ds(s * TS, TS)],
                        out_hbm.at[c, 0, pl.ds(s * TS, TS)])

    return k


def _sc_row_scatter(E, N, NR, D):
    EW = E // NW
    CH = 125
    NCH = EW // CH
    RT = NR // NS
    mesh = plsc.VectorSubcoreMesh(core_axis_name="c", subcore_axis_name="s")

    @functools.partial(
        pl.kernel,
        out_type=jax.ShapeDtypeStruct((NC, NR, D), jnp.float32),
        mesh=mesh,
        scratch_types=[
            pltpu.VMEM((NCH, CH), jnp.int32),
            pltpu.VMEM((NCH, CH), jnp.int32),
            pltpu.VMEM((CH, D), jnp.float32),
            pltpu.VMEM_SHARED((NR, D), jnp.float32),
            pltpu.SemaphoreType.DMA,
        ],
        compiler_params=pltpu.CompilerParams(
            use_tc_tiling_on_sc=False, needs_layout_passes=False),
    )
    def k(src_hbm, dst_hbm, tab_hbm, zrows_hbm, out_hbm,
          src_buf, dst_buf, rows, acc, sem):
        c = lax.axis_index("c")
        s = lax.axis_index("s")
        w = c * NS + s
        pltpu.sync_copy(zrows_hbm.at[pl.ds(s * RT, RT)], acc.at[pl.ds(s * RT, RT)])
        pltpu.sync_copy(src_hbm.at[w], src_buf)
        pltpu.sync_copy(dst_hbm.at[w], dst_buf)
        plsc.subcore_barrier()

        def body(j, carry):
            pltpu.async_copy(tab_hbm.at[src_buf.at[j]], rows, sem).wait()
            pltpu.sync_copy(rows, acc.at[dst_buf.at[j]], add=True)
            return carry

        lax.fori_loop(0, NCH, body, 0)
        plsc.subcore_barrier()
        pltpu.sync_copy(acc.at[pl.ds(s * RT, RT)], out_hbm.at[c, pl.ds(s * RT, RT)])

    return k


def _tc_scale(N, D, H, B):
    def body(x_b, w1_b, d0_b, d1_b, dinv_b, xws_b):
        deg = d0_b[...] + d1_b[...] + 1.0
        dinv = jnp.where(deg > 0, lax.rsqrt(deg), 0.0)
        dinv_b[...] = dinv
        xw = jnp.dot(x_b[...], w1_b[...], preferred_element_type=jnp.float32)
        xws_b[...] = dinv * xw

    return pl.pallas_call(
        body,
        grid=(N // B,),
        in_specs=[
            pl.BlockSpec((B, D), lambda i: (i, 0)),
            pl.BlockSpec((D, H), lambda i: (0, 0)),
            pl.BlockSpec((B, 1), lambda i: (i, 0)),
            pl.BlockSpec((B, 1), lambda i: (i, 0)),
        ],
        out_specs=[
            pl.BlockSpec((B, 1), lambda i: (i, 0)),
            pl.BlockSpec((B, H), lambda i: (i, 0)),
        ],
        out_shape=[
            jax.ShapeDtypeStruct((N, 1), jnp.float32),
            jax.ShapeDtypeStruct((N, H), jnp.float32),
        ],
    )


def _tc_layer1(N, H, B):
    def body(a0_b, a1_b, xws_b, dinv_b, b1_b, w2_b, zs_b):
        dinv = dinv_b[...]
        h1 = dinv * (a0_b[...] + a1_b[...] + xws_b[...]) + b1_b[...]
        h1 = jnp.maximum(h1, 0.0)
        z = jnp.dot(h1, w2_b[...], preferred_element_type=jnp.float32)
        zs_b[...] = dinv * z

    return pl.pallas_call(
        body,
        grid=(N // B,),
        in_specs=[
            pl.BlockSpec((B, H), lambda i: (i, 0)),
            pl.BlockSpec((B, H), lambda i: (i, 0)),
            pl.BlockSpec((B, H), lambda i: (i, 0)),
            pl.BlockSpec((B, 1), lambda i: (i, 0)),
            pl.BlockSpec((1, H), lambda i: (0, 0)),
            pl.BlockSpec((H, 1), lambda i: (0, 0)),
        ],
        out_specs=pl.BlockSpec((B, 1), lambda i: (i, 0)),
        out_shape=jax.ShapeDtypeStruct((N, 1), jnp.float32),
    )


def _tc_final(N, B):
    def body(c0_b, c1_b, zs_b, dinv_b, b2_b, xl_b, o_b):
        o = dinv_b[...] * (c0_b[...] + c1_b[...] + zs_b[...]) + b2_b[...]
        o_b[...] = jnp.maximum(o, 0.0) + xl_b[...]

    col = lambda i: (i, 0)
    return pl.pallas_call(
        body,
        grid=(N // B,),
        in_specs=[
            pl.BlockSpec((B, 1), col),
            pl.BlockSpec((B, 1), col),
            pl.BlockSpec((B, 1), col),
            pl.BlockSpec((B, 1), col),
            pl.BlockSpec((1, 1), lambda i: (0, 0)),
            pl.BlockSpec((B, 1), col),
        ],
        out_specs=pl.BlockSpec((B, 1), col),
        out_shape=jax.ShapeDtypeStruct((N, 1), jnp.float32),
    )


def kernel(x, edge_index, batch, W1, b1, W2, b2):
    N, D = x.shape
    H = W1.shape[1]
    E = edge_index.shape[1]
    NP = ((N + 128 * NS - 1) // (128 * NS)) * (128 * NS)
    NR = NP
    B = N // 10
    EW = E // NW

    src = edge_index[0]
    dst = edge_index[1]
    src3s = src.reshape(NW, EW // 80, 80)
    dst3s = dst.reshape(NW, EW // 80, 80)
    src3r = src.reshape(NW, EW // 125, 125)
    dst3r = dst.reshape(NW, EW // 125, 125)

    ones_n = jnp.ones((N,), jnp.float32)
    zvec = jnp.zeros((NP,), jnp.float32)
    zrows = jnp.zeros((NR, D), jnp.float32)

    scalar_scatter = _sc_scalar_scatter(E, N, NP)
    row_scatter = _sc_row_scatter(E, N, NR, D)

    degp = scalar_scatter(src3s, dst3s, ones_n, zvec)
    d0 = degp[0, 0, :N].reshape(N, 1)
    d1 = degp[1, 0, :N].reshape(N, 1)

    dinv, xws = _tc_scale(N, D, H, B)(x, W1, d0, d1)

    accp = row_scatter(src3r, dst3r, xws, zrows)

    zs = _tc_layer1(N, H, B)(
        accp[0, :N], accp[1, :N], xws, dinv, b1.reshape(1, H), W2)

    acc2p = scalar_scatter(src3s, dst3s, zs.reshape(N), zvec)
    c0 = acc2p[0, 0, :N].reshape(N, 1)
    c1 = acc2p[1, 0, :N].reshape(N, 1)

    out_col = _tc_final(N, B)(
        c0, c1, zs, dinv, b2.reshape(1, 1), x[:, -1:])
    return out_col.reshape(1, N)


# speedup vs baseline: 31.7772x; 1.1496x over previous
import functools

import jax
import jax.numpy as jnp
from jax import lax
from jax.experimental import pallas as pl
from jax.experimental.pallas import tpu as pltpu
from jax.experimental.pallas import tpu_sc as plsc

NC = 2
NS = 16
NW = NC * NS


def _sc_scalar_scatter(E, N, NP, count_only):
    EW = E // NW
    CH = 80
    NCH = EW // CH
    TS = NP // NS
    mesh = plsc.VectorSubcoreMesh(core_axis_name="c", subcore_axis_name="s")

    @functools.partial(
        pl.kernel,
        out_type=jax.ShapeDtypeStruct((NC, 1, NP), jnp.float32),
        mesh=mesh,
        scratch_types=[
            pltpu.VMEM((NCH, CH), jnp.int32),
            pltpu.VMEM((NCH, CH), jnp.int32),
            pltpu.VMEM((N,), jnp.float32),
            pltpu.VMEM((NCH, CH), jnp.float32),
            pltpu.VMEM_SHARED((NP,), jnp.float32),
            pltpu.SemaphoreType.DMA,
        ],
        compiler_params=pltpu.CompilerParams(
            use_tc_tiling_on_sc=False, needs_layout_passes=False),
    )
    def k(src_hbm, dst_hbm, val_hbm, zvec_hbm, out_hbm,
          src_buf, dst_buf, val_tab, vals_buf, acc, sem):
        c = lax.axis_index("c")
        s = lax.axis_index("s")
        w = c * NS + s
        pltpu.sync_copy(zvec_hbm.at[pl.ds(s * TS, TS)], acc.at[pl.ds(s * TS, TS)])
        pltpu.sync_copy(dst_hbm.at[w], dst_buf)

        if count_only:
            ones16 = jnp.full((16,), 1.0, jnp.float32)
            for u in range(CH // 16):
                vals_buf[0, pl.ds(u * 16, 16)] = ones16
        else:
            pltpu.sync_copy(val_hbm, val_tab)
            pltpu.sync_copy(src_hbm.at[w], src_buf)

            def gather_body(j, carry):
                for u in range(CH // 16):
                    idx = src_buf[j, pl.ds(u * 16, 16)]
                    vals_buf[j, pl.ds(u * 16, 16)] = plsc.load_gather(val_tab, [idx])
                return carry

            lax.fori_loop(0, NCH, gather_body, 0)
        plsc.subcore_barrier()

        def scatter_body(j, carry):
            jv = 0 if count_only else j
            pltpu.sync_copy(vals_buf.at[jv], acc.at[dst_buf.at[j]], add=True)
            return carry

        lax.fori_loop(0, NCH, scatter_body, 0)
        plsc.subcore_barrier()
        pltpu.sync_copy(acc.at[pl.ds(s * TS, TS)],
                        out_hbm.at[c, 0, pl.ds(s * TS, TS)])

    return k


def _sc_row_scatter(E, N, NR, D):
    EW = E // NW
    CH = 100
    NCH = EW // CH
    RT = NR // NS
    mesh = plsc.VectorSubcoreMesh(core_axis_name="c", subcore_axis_name="s")

    @functools.partial(
        pl.kernel,
        out_type=jax.ShapeDtypeStruct((NC, NR, D), jnp.float32),
        mesh=mesh,
        scratch_types=[
            pltpu.VMEM((NCH, CH), jnp.int32),
            pltpu.VMEM((NCH, CH), jnp.int32),
            pltpu.VMEM((CH, D), jnp.float32),
            pltpu.VMEM((CH, D), jnp.float32),
            pltpu.VMEM_SHARED((NR, D), jnp.float32),
            pltpu.SemaphoreType.DMA,
            pltpu.SemaphoreType.DMA,
        ],
        compiler_params=pltpu.CompilerParams(
            use_tc_tiling_on_sc=False, needs_layout_passes=False),
    )
    def k(src_hbm, dst_hbm, tab_hbm, zrows_hbm, out_hbm,
          src_buf, dst_buf, rows0, rows1, acc, sem0, sem1):
        c = lax.axis_index("c")
        s = lax.axis_index("s")
        w = c * NS + s
        pltpu.sync_copy(src_hbm.at[w], src_buf)
        pltpu.sync_copy(dst_hbm.at[w], dst_buf)
        pltpu.async_copy(tab_hbm.at[src_buf.at[0]], rows0, sem0)
        pltpu.sync_copy(zrows_hbm.at[pl.ds(s * RT, RT)], acc.at[pl.ds(s * RT, RT)])
        plsc.subcore_barrier()

        def body(i, carry):
            j0 = 2 * i
            j1 = 2 * i + 1
            pltpu.make_async_copy(tab_hbm.at[src_buf.at[j0]], rows0, sem0).wait()
            pltpu.async_copy(tab_hbm.at[src_buf.at[j1]], rows1, sem1)
            pltpu.sync_copy(rows0, acc.at[dst_buf.at[j0]], add=True)
            pltpu.make_async_copy(tab_hbm.at[src_buf.at[j1]], rows1, sem1).wait()
            jn = lax.min(j0 + 2, NCH - 1)
            pltpu.async_copy(tab_hbm.at[src_buf.at[jn]], rows0, sem0)
            pltpu.sync_copy(rows1, acc.at[dst_buf.at[j1]], add=True)
            return carry

        lax.fori_loop(0, NCH // 2, body, 0)
        pltpu.make_async_copy(tab_hbm.at[src_buf.at[NCH - 1]], rows0, sem0).wait()
        plsc.subcore_barrier()
        pltpu.sync_copy(acc.at[pl.ds(s * RT, RT)], out_hbm.at[c, pl.ds(s * RT, RT)])

    return k


def _tc_scale(N, D, H, B):
    def body(x_b, w1_b, d0_b, d1_b, dinv_b, xws_b):
        deg = d0_b[...] + d1_b[...] + 1.0
        dinv = jnp.where(deg > 0, lax.rsqrt(deg), 0.0)
        dinv_b[...] = dinv
        xw = jnp.dot(x_b[...], w1_b[...], preferred_element_type=jnp.float32)
        xws_b[...] = dinv * xw

    return pl.pallas_call(
        body,
        grid=(N // B,),
        in_specs=[
            pl.BlockSpec((B, D), lambda i: (i, 0)),
            pl.BlockSpec((D, H), lambda i: (0, 0)),
            pl.BlockSpec((B, 1), lambda i: (i, 0)),
            pl.BlockSpec((B, 1), lambda i: (i, 0)),
        ],
        out_specs=[
            pl.BlockSpec((B, 1), lambda i: (i, 0)),
            pl.BlockSpec((B, H), lambda i: (i, 0)),
        ],
        out_shape=[
            jax.ShapeDtypeStruct((N, 1), jnp.float32),
            jax.ShapeDtypeStruct((N, H), jnp.float32),
        ],
    )


def _tc_layer1(N, H, B):
    def body(a0_b, a1_b, xws_b, dinv_b, b1_b, w2_b, zs_b):
        dinv = dinv_b[...]
        h1 = dinv * (a0_b[...] + a1_b[...] + xws_b[...]) + b1_b[...]
        h1 = jnp.maximum(h1, 0.0)
        z = jnp.dot(h1, w2_b[...], preferred_element_type=jnp.float32)
        zs_b[...] = dinv * z

    return pl.pallas_call(
        body,
        grid=(N // B,),
        in_specs=[
            pl.BlockSpec((B, H), lambda i: (i, 0)),
            pl.BlockSpec((B, H), lambda i: (i, 0)),
            pl.BlockSpec((B, H), lambda i: (i, 0)),
            pl.BlockSpec((B, 1), lambda i: (i, 0)),
            pl.BlockSpec((1, H), lambda i: (0, 0)),
            pl.BlockSpec((H, 1), lambda i: (0, 0)),
        ],
        out_specs=pl.BlockSpec((B, 1), lambda i: (i, 0)),
        out_shape=jax.ShapeDtypeStruct((N, 1), jnp.float32),
    )


def _tc_final(N, B):
    def body(c0_b, c1_b, zs_b, dinv_b, b2_b, xl_b, o_b):
        o = dinv_b[...] * (c0_b[...] + c1_b[...] + zs_b[...]) + b2_b[...]
        o_b[...] = jnp.maximum(o, 0.0) + xl_b[...]

    col = lambda i: (i, 0)
    return pl.pallas_call(
        body,
        grid=(N // B,),
        in_specs=[
            pl.BlockSpec((B, 1), col),
            pl.BlockSpec((B, 1), col),
            pl.BlockSpec((B, 1), col),
            pl.BlockSpec((B, 1), col),
            pl.BlockSpec((1, 1), lambda i: (0, 0)),
            pl.BlockSpec((B, 1), col),
        ],
        out_specs=pl.BlockSpec((B, 1), col),
        out_shape=jax.ShapeDtypeStruct((N, 1), jnp.float32),
    )


def kernel(x, edge_index, batch, W1, b1, W2, b2):
    N, D = x.shape
    H = W1.shape[1]
    E = edge_index.shape[1]
    NP = ((N + 128 * NS - 1) // (128 * NS)) * (128 * NS)
    NR = NP
    B = N // 10
    EW = E // NW

    src = edge_index[0]
    dst = edge_index[1]
    src3s = src.reshape(NW, EW // 80, 80)
    dst3s = dst.reshape(NW, EW // 80, 80)
    src3r = src.reshape(NW, EW // 100, 100)
    dst3r = dst.reshape(NW, EW // 100, 100)

    ones_n = jnp.ones((N,), jnp.float32)
    zvec = jnp.zeros((NP,), jnp.float32)
    zrows = jnp.zeros((NR, D), jnp.float32)

    scalar_scatter = _sc_scalar_scatter(E, N, NP, count_only=False)
    count_scatter = _sc_scalar_scatter(E, N, NP, count_only=True)
    row_scatter = _sc_row_scatter(E, N, NR, D)

    degp = count_scatter(src3s, dst3s, ones_n, zvec)
    d0 = degp[0, 0, :N].reshape(N, 1)
    d1 = degp[1, 0, :N].reshape(N, 1)

    dinv, xws = _tc_scale(N, D, H, B)(x, W1, d0, d1)

    accp = row_scatter(src3r, dst3r, xws, zrows)

    zs = _tc_layer1(N, H, B)(
        accp[0, :N], accp[1, :N], xws, dinv, b1.reshape(1, H), W2)

    acc2p = scalar_scatter(src3s, dst3s, zs.reshape(N), zvec)
    c0 = acc2p[0, 0, :N].reshape(N, 1)
    c1 = acc2p[1, 0, :N].reshape(N, 1)

    out_col = _tc_final(N, B)(
        c0, c1, zs, dinv, b2.reshape(1, 1), x[:, -1:])
    return out_col.reshape(1, N)
